# baseline (speedup 1.0000x reference)
import functools

import jax
import jax.numpy as jnp
import numpy as np
from jax import lax
from jax.experimental import pallas as pl
from jax.experimental.pallas import tpu as pltpu
from jax.experimental.pallas import tpu_sc as plsc

_NX = 64
_NY = 64
_B = 32
_NSPOT = 256
_P = 6
_PHW = 3
_FC = 65536
_H1 = 256
_H2 = 128
_LAT = 512
_NVAL = 48

_t = (np.arange(5, dtype=np.float32) - np.float32(2.0))
_k1 = np.exp(np.float32(-0.5) * (_t / np.float32(1.0)) ** 2).astype(np.float32)
_k1 = (_k1 / _k1.sum(dtype=np.float32)).astype(np.float32)
_W2D = np.outer(_k1, _k1).astype(np.float32)

_ALPHA = float(np.float32(np.sqrt(np.float32(2.0))) * np.float32(0.92))
_I0 = 1000.0

import ml_dtypes as _mld

_KCNP = np.zeros((5, 260, 256), np.float32)
for _di in range(5):
    for _dj in range(5):
        _KCNP[_di, _dj + np.arange(256), np.arange(256)] = _W2D[_di, _dj]
_KC_BF = _KCNP.astype(_mld.bfloat16)


_BIMG = 8


def _blur_body(x_ref, kc_ref, o_ref, oa_ref):
    dn = (((1,), (0,)), ((), ()))
    for b in range(_BIMG):
        xf = x_ref[b]
        xc = xf.astype(jnp.bfloat16)
        xw = jnp.concatenate([xc[:, 2:3], xc[:, 1:2], xc,
                              xc[:, 254:255], xc[:, 253:254]], axis=1)
        acc = None
        for di in range(5):
            lo = di - 2
            if lo < 0:
                pieces = [xw[-lo - k:-lo - k + 1] for k in range(-lo)]
                pieces.append(xw[0:256 + lo])
            elif lo == 0:
                pieces = [xw]
            else:
                pieces = [xw[lo:256]]
                pieces.extend(xw[254 - k:255 - k] for k in range(lo))
            a = jnp.concatenate(pieces, axis=0) if len(pieces) > 1 else pieces[0]
            d = lax.dot_general(a, kc_ref[di], dn,
                                preferred_element_type=jnp.float32)
            acc = d if acc is None else acc + d
        o_ref[b] = acc
        oa_ref[b] = acc.reshape(_FC)


_KCHUNK = 16
_ROWS = 256 // _KCHUNK


def _mega_body(x_ref, kc_ref, wmu_ref, wlv_ref,
               bmu1_ref, blv1_ref, wmu2_ref, bmu2_ref, wmu3_ref, bmu3_ref,
               wlv2_ref, blv2_ref, wlv3_ref, blv3_ref, eps_ref,
               conv_ref, mu_ref, lv_ref, vals_ref, idx_ref,
               xbp_ref, actc_ref, amu_ref, alv_ref):
    i = pl.program_id(0)

    @pl.when(i == 0)
    def _prep_images():
        for b in range(_B):
            xc = x_ref[b].astype(jnp.bfloat16)
            xw = jnp.concatenate([xc[:, 2:3], xc[:, 1:2], xc,
                                  xc[:, 254:255], xc[:, 253:254]], axis=1)
            xbp_ref[b] = jnp.concatenate([xw[2:3], xw[1:2], xw,
                                          xw[254:255], xw[253:254],
                                          jnp.zeros((12, 260), jnp.bfloat16)],
                                         axis=0)

    dn = (((1,), (0,)), ((), ()))
    base = pl.multiple_of(i * _ROWS, _ROWS)
    for b in range(_B):
        slab = xbp_ref[b, pl.ds(base, 32), :]
        acc = None
        for di in range(5):
            a = slab[di:di + _ROWS, :]
            d = lax.dot_general(a, kc_ref[di], dn,
                                preferred_element_type=jnp.float32)
            acc = d if acc is None else acc + d
        conv_ref[b] = acc
        actc_ref[b] = acc.reshape(_ROWS * 256)

    dnt = (((1,), (1,)), ((), ()))
    a2 = actc_ref[...]
    hmu = lax.dot_general(a2, wmu_ref[...], dnt,
                          preferred_element_type=jnp.float32)
    hlv = lax.dot_general(a2, wlv_ref[...], dnt,
                          preferred_element_type=jnp.float32)

    @pl.when(i == 0)
    def _init():
        amu_ref[...] = hmu
        alv_ref[...] = hlv

    @pl.when(i > 0)
    def _accum():
        amu_ref[...] += hmu
        alv_ref[...] += hlv

    @pl.when(i == _KCHUNK - 1)
    def _epilogue():
        dnc = (((1,), (1,)), ((), ()))

        def head(hpre, b1, w2, b2, w3, b3):
            h1 = jnp.maximum(hpre + b1, 0.0)
            h2 = jnp.maximum(
                lax.dot_general(h1, w2, dnc,
                                preferred_element_type=jnp.float32) + b2, 0.0)
            return (lax.dot_general(h2, w3, dnc,
                                    preferred_element_type=jnp.float32) + b3)

        mu = head(amu_ref[...], bmu1_ref[...], wmu2_ref[...], bmu2_ref[...],
                  wmu3_ref[...], bmu3_ref[...])
        lv = head(alv_ref[...], blv1_ref[...], wlv2_ref[...], blv2_ref[...],
                  wlv3_ref[...], blv3_ref[...])
        mu_ref[...] = mu
        lv_ref[...] = lv
        std = jnp.exp(0.5 * lv)
        z = mu + eps_ref[...] * std + (_NX / 2.0)
        x0 = z[:, :_NSPOT].reshape(1, _B * _NSPOT)
        y0 = z[:, _NSPOT:].reshape(1, _B * _NSPOT)
        _psf_prep(x0, y0, vals_ref, idx_ref)


def _head_body(hmu_ref, hlv_ref, bmu1_ref, blv1_ref, wmu2_ref, bmu2_ref,
               wmu3_ref, bmu3_ref, wlv2_ref, blv2_ref, wlv3_ref, blv3_ref,
               eps_ref, mu_ref, lv_ref, vals_ref, idx_ref):
    dn = (((1,), (1,)), ((), ()))

    def head(hpre, b1, w2, b2, w3, b3):
        h1 = jnp.maximum(hpre + b1, 0.0)
        h2 = jnp.maximum(
            lax.dot_general(h1, w2, dn, preferred_element_type=jnp.float32)
            + b2, 0.0)
        return (lax.dot_general(h2, w3, dn, preferred_element_type=jnp.float32)
                + b3)

    mu = head(hmu_ref[...], bmu1_ref[...], wmu2_ref[...], bmu2_ref[...],
              wmu3_ref[...], bmu3_ref[...])
    lv = head(hlv_ref[...], blv1_ref[...], wlv2_ref[...], blv2_ref[...],
              wlv3_ref[...], blv3_ref[...])
    mu_ref[...] = mu
    lv_ref[...] = lv
    std = jnp.exp(0.5 * lv)
    z = mu + eps_ref[...] * std + (_NX / 2.0)
    x0 = z[:, :_NSPOT].reshape(1, _B * _NSPOT)
    y0 = z[:, _NSPOT:].reshape(1, _B * _NSPOT)
    _psf_prep(x0, y0, vals_ref, idx_ref)


def _psf_prep(x0, y0, vals_ref, idx_ref):
    x0r = jnp.round(x0).astype(jnp.int32)
    y0r = jnp.round(y0).astype(jnp.int32)
    px = x0r - _PHW
    py = y0r - _PHW
    x0p = x0 - px.astype(jnp.float32)
    y0p = y0 - py.astype(jnp.float32)
    lim = _NX - _P
    mask = ((px >= 0) & (px < lim) & (py >= 0) & (py < lim))
    m = mask.astype(jnp.float32)
    pxc = jnp.clip(px, 0, lim)
    pyc = jnp.clip(py, 0, lim)

    def erf_edge(c, ctr):
        return jax.scipy.special.erf((c - ctr) / _ALPHA)

    lxs = [0.5 * (erf_edge(i + 0.5, x0p) - erf_edge(i - 0.5, x0p))
           for i in range(_P)]
    lys = [0.5 * (erf_edge(j + 0.5, y0p) - erf_edge(j - 0.5, y0p))
           for j in range(_P)]
    ly_stack = jnp.concatenate(lys, axis=0)
    jdx = lax.broadcasted_iota(jnp.int32, (_P, 8192), 0)

    val_rows = []
    idx_rows = []
    for i in range(_P):
        val_rows.append(((_I0 * lxs[i]) * ly_stack) * m)
        idx_rows.append((pxc + i) * _NY + pyc + jdx)
    val_rows.append(jnp.zeros((_NVAL - _P * _P, 8192), jnp.float32))
    idx_rows.append(_NX * _NY
                    + lax.broadcasted_iota(jnp.int32, (_NVAL - _P * _P, 8192), 0))
    vals_ref[...] = jnp.concatenate(val_rows, axis=0).T
    idx_ref[...] = jnp.concatenate(idx_rows, axis=0).T


def _scatter_images(vals, idx):
    mesh = plsc.VectorSubcoreMesh(core_axis_name="c", subcore_axis_name="s",
                                  num_cores=2)

    @functools.partial(
        pl.kernel,
        out_type=jax.ShapeDtypeStruct((_B, _NX * _NY), jnp.float32),
        mesh=mesh,
        compiler_params=pltpu.CompilerParams(needs_layout_passes=False),
        scratch_types=[
            pltpu.VMEM((_NSPOT, _NVAL), jnp.float32),
            pltpu.VMEM((_NSPOT, _NVAL), jnp.int32),
            pltpu.VMEM((_NX * _NY + 16,), jnp.float32),
            pltpu.SemaphoreType.DMA,
            pltpu.SemaphoreType.DMA,
        ],
    )
    def k(vals_hbm, idx_hbm, out_hbm, vals_v, idx_v, img_v, sem_v, sem_i):
        b = lax.axis_index("s") * 2 + lax.axis_index("c")
        cp_v = pltpu.async_copy(vals_hbm.at[pl.ds(b * _NSPOT, _NSPOT)],
                                vals_v, sem_v)
        cp_i = pltpu.async_copy(idx_hbm.at[pl.ds(b * _NSPOT, _NSPOT)],
                                idx_v, sem_i)

        @pl.loop(0, _NX * _NY + 16, step=16)
        def _zero(i):
            img_v[pl.ds(i, 16)] = jnp.zeros((16,), jnp.float32)

        cp_v.wait()
        cp_i.wait()

        @plsc.parallel_loop(0, _NSPOT, unroll=4)
        def _spot(s):
            for c in range(_NVAL // 16):
                ix = idx_v[s, pl.ds(c * 16, 16)]
                v = vals_v[s, pl.ds(c * 16, 16)]
                plsc.addupdate_scatter(img_v, [ix], v)

        pltpu.sync_copy(img_v.at[pl.ds(0, _NX * _NY)], out_hbm.at[b])

    return k(vals, idx)


def kernel(input, eps, W_mu1, b_mu1, W_mu2, b_mu2, W_mu3, b_mu3,
           W_lv1, b_lv1, W_lv2, b_lv2, W_lv3, b_lv3):
    x = input[:, 0]
    kblk = _FC // _KCHUNK
    conv, mu, logvar, vals, idxs = pl.pallas_call(
        _mega_body,
        grid=(_KCHUNK,),
        in_specs=[
            pl.BlockSpec((_B, 256, 256), lambda i: (0, 0, 0)),
            pl.BlockSpec((5, 260, 256), lambda i: (0, 0, 0)),
            pl.BlockSpec((_H1, kblk), lambda i: (0, i)),
            pl.BlockSpec((_H1, kblk), lambda i: (0, i)),
            pl.BlockSpec((1, _H1), lambda i: (0, 0)),
            pl.BlockSpec((1, _H1), lambda i: (0, 0)),
            pl.BlockSpec((_H2, _H1), lambda i: (0, 0)),
            pl.BlockSpec((1, _H2), lambda i: (0, 0)),
            pl.BlockSpec((_LAT, _H2), lambda i: (0, 0)),
            pl.BlockSpec((1, _LAT), lambda i: (0, 0)),
            pl.BlockSpec((_H2, _H1), lambda i: (0, 0)),
            pl.BlockSpec((1, _H2), lambda i: (0, 0)),
            pl.BlockSpec((_LAT, _H2), lambda i: (0, 0)),
            pl.BlockSpec((1, _LAT), lambda i: (0, 0)),
            pl.BlockSpec((_B, _LAT), lambda i: (0, 0)),
        ],
        out_specs=[
            pl.BlockSpec((_B, _ROWS, 256), lambda i: (0, i, 0)),
            pl.BlockSpec((_B, _LAT), lambda i: (0, 0)),
            pl.BlockSpec((_B, _LAT), lambda i: (0, 0)),
            pl.BlockSpec((_B * _NSPOT, _NVAL), lambda i: (0, 0)),
            pl.BlockSpec((_B * _NSPOT, _NVAL), lambda i: (0, 0)),
        ],
        out_shape=[jax.ShapeDtypeStruct((_B, 256, 256), jnp.float32),
                   jax.ShapeDtypeStruct((_B, _LAT), jnp.float32),
                   jax.ShapeDtypeStruct((_B, _LAT), jnp.float32),
                   jax.ShapeDtypeStruct((_B * _NSPOT, _NVAL), jnp.float32),
                   jax.ShapeDtypeStruct((_B * _NSPOT, _NVAL), jnp.int32)],
        scratch_shapes=[
            pltpu.VMEM((_B, 272, 260), jnp.bfloat16),
            pltpu.VMEM((_B, _ROWS * 256), jnp.float32),
            pltpu.VMEM((_B, _H1), jnp.float32),
            pltpu.VMEM((_B, _H1), jnp.float32),
        ],
    )(x, jnp.asarray(_KC_BF), W_mu1, W_lv1,
      b_mu1.reshape(1, _H1), b_lv1.reshape(1, _H1),
      W_mu2, b_mu2.reshape(1, _H2), W_mu3, b_mu3.reshape(1, _LAT),
      W_lv2, b_lv2.reshape(1, _H2), W_lv3, b_lv3.reshape(1, _LAT), eps)

    img = _scatter_images(vals, idxs)
    return (img.reshape(_B, 1, _NX, _NY), conv.reshape(_B, 1, 256, 256),
            mu, logvar)


# speedup vs baseline: 1.3387x; 1.3387x over previous
import functools

import jax
import jax.numpy as jnp
import numpy as np
from jax import lax
from jax.experimental import pallas as pl
from jax.experimental.pallas import tpu as pltpu
from jax.experimental.pallas import tpu_sc as plsc

_NX = 64
_NY = 64
_B = 32
_NSPOT = 256
_P = 6
_PHW = 3
_FC = 65536
_H1 = 256
_H2 = 128
_LAT = 512
_NVAL = 48

_t = (np.arange(5, dtype=np.float32) - np.float32(2.0))
_k1 = np.exp(np.float32(-0.5) * (_t / np.float32(1.0)) ** 2).astype(np.float32)
_k1 = (_k1 / _k1.sum(dtype=np.float32)).astype(np.float32)
_W2D = np.outer(_k1, _k1).astype(np.float32)

_ALPHA = float(np.float32(np.sqrt(np.float32(2.0))) * np.float32(0.92))
_I0 = 1000.0

import ml_dtypes as _mld

_KCNP = np.zeros((5, 260, 256), np.float32)
for _di in range(5):
    for _dj in range(5):
        _KCNP[_di, _dj + np.arange(256), np.arange(256)] = _W2D[_di, _dj]
_KC_BF = _KCNP.astype(_mld.bfloat16)


_BIMG = 8


def _blur_body(x_ref, kc_ref, o_ref, oa_ref):
    dn = (((1,), (0,)), ((), ()))
    for b in range(_BIMG):
        xf = x_ref[b]
        xc = xf.astype(jnp.bfloat16)
        xw = jnp.concatenate([xc[:, 2:3], xc[:, 1:2], xc,
                              xc[:, 254:255], xc[:, 253:254]], axis=1)
        acc = None
        for di in range(5):
            lo = di - 2
            if lo < 0:
                pieces = [xw[-lo - k:-lo - k + 1] for k in range(-lo)]
                pieces.append(xw[0:256 + lo])
            elif lo == 0:
                pieces = [xw]
            else:
                pieces = [xw[lo:256]]
                pieces.extend(xw[254 - k:255 - k] for k in range(lo))
            a = jnp.concatenate(pieces, axis=0) if len(pieces) > 1 else pieces[0]
            d = lax.dot_general(a, kc_ref[di], dn,
                                preferred_element_type=jnp.float32)
            acc = d if acc is None else acc + d
        o_ref[b] = acc
        oa_ref[b] = acc.reshape(_FC)


_NBLK = 8


def _mm_body(act_ref, wmu_ref, wlv_ref,
             bmu1_ref, blv1_ref, wmu2_ref, bmu2_ref, wmu3_ref, bmu3_ref,
             wlv2_ref, blv2_ref, wlv3_ref, blv3_ref, eps_ref,
             mu_ref, lv_ref, vals_ref, idx_ref, h_ref):
    i = pl.program_id(0)
    a = act_ref[...]
    dn = (((1,), (1,)), ((), ()))
    w = jnp.concatenate([wmu_ref[...], wlv_ref[...]], axis=0)
    h_ref[i] = lax.dot_general(a, w, dn, preferred_element_type=jnp.float32)

    @pl.when(i == _NBLK - 1)
    def _epilogue():
        fb = _H1 // _NBLK
        hmu = jnp.concatenate([h_ref[j][:, :fb] for j in range(_NBLK)], axis=1)
        hlv = jnp.concatenate([h_ref[j][:, fb:] for j in range(_NBLK)], axis=1)

        def head(hpre, b1, w2, b2, w3, b3):
            h1 = jnp.maximum(hpre + b1, 0.0)
            h2 = jnp.maximum(
                lax.dot_general(h1, w2, dn,
                                preferred_element_type=jnp.float32) + b2, 0.0)
            return (lax.dot_general(h2, w3, dn,
                                    preferred_element_type=jnp.float32) + b3)

        mu = head(hmu, bmu1_ref[...], wmu2_ref[...], bmu2_ref[...],
                  wmu3_ref[...], bmu3_ref[...])
        lv = head(hlv, blv1_ref[...], wlv2_ref[...], blv2_ref[...],
                  wlv3_ref[...], blv3_ref[...])
        mu_ref[...] = mu
        lv_ref[...] = lv
        std = jnp.exp(0.5 * lv)
        z = mu + eps_ref[...] * std + (_NX / 2.0)
        x0 = z[:, :_NSPOT].reshape(1, _B * _NSPOT)
        y0 = z[:, _NSPOT:].reshape(1, _B * _NSPOT)
        _psf_prep(x0, y0, vals_ref, idx_ref)


def _head_body(hmu_ref, hlv_ref, bmu1_ref, blv1_ref, wmu2_ref, bmu2_ref,
               wmu3_ref, bmu3_ref, wlv2_ref, blv2_ref, wlv3_ref, blv3_ref,
               eps_ref, mu_ref, lv_ref, vals_ref, idx_ref):
    dn = (((1,), (1,)), ((), ()))

    def head(hpre, b1, w2, b2, w3, b3):
        h1 = jnp.maximum(hpre + b1, 0.0)
        h2 = jnp.maximum(
            lax.dot_general(h1, w2, dn, preferred_element_type=jnp.float32)
            + b2, 0.0)
        return (lax.dot_general(h2, w3, dn, preferred_element_type=jnp.float32)
                + b3)

    mu = head(hmu_ref[...], bmu1_ref[...], wmu2_ref[...], bmu2_ref[...],
              wmu3_ref[...], bmu3_ref[...])
    lv = head(hlv_ref[...], blv1_ref[...], wlv2_ref[...], blv2_ref[...],
              wlv3_ref[...], blv3_ref[...])
    mu_ref[...] = mu
    lv_ref[...] = lv
    std = jnp.exp(0.5 * lv)
    z = mu + eps_ref[...] * std + (_NX / 2.0)
    x0 = z[:, :_NSPOT].reshape(1, _B * _NSPOT)
    y0 = z[:, _NSPOT:].reshape(1, _B * _NSPOT)
    _psf_prep(x0, y0, vals_ref, idx_ref)


def _psf_prep(x0, y0, vals_ref, idx_ref):
    x0r = jnp.round(x0).astype(jnp.int32)
    y0r = jnp.round(y0).astype(jnp.int32)
    px = x0r - _PHW
    py = y0r - _PHW
    x0p = x0 - px.astype(jnp.float32)
    y0p = y0 - py.astype(jnp.float32)
    lim = _NX - _P
    mask = ((px >= 0) & (px < lim) & (py >= 0) & (py < lim))
    m = mask.astype(jnp.float32)
    pxc = jnp.clip(px, 0, lim)
    pyc = jnp.clip(py, 0, lim)

    def erf_edge(c, ctr):
        return jax.scipy.special.erf((c - ctr) / _ALPHA)

    lxs = [0.5 * (erf_edge(i + 0.5, x0p) - erf_edge(i - 0.5, x0p))
           for i in range(_P)]
    lys = [0.5 * (erf_edge(j + 0.5, y0p) - erf_edge(j - 0.5, y0p))
           for j in range(_P)]
    ly_stack = jnp.concatenate(lys, axis=0)
    jdx = lax.broadcasted_iota(jnp.int32, (_P, 8192), 0)

    val_rows = []
    idx_rows = []
    for i in range(_P):
        val_rows.append(((_I0 * lxs[i]) * ly_stack) * m)
        idx_rows.append((pxc + i) * _NY + pyc + jdx)
    val_rows.append(jnp.zeros((_NVAL - _P * _P, 8192), jnp.float32))
    idx_rows.append(_NX * _NY
                    + lax.broadcasted_iota(jnp.int32, (_NVAL - _P * _P, 8192), 0))
    vals_ref[...] = jnp.concatenate(val_rows, axis=0).T
    idx_ref[...] = jnp.concatenate(idx_rows, axis=0).T


def _scatter_images(vals, idx):
    mesh = plsc.VectorSubcoreMesh(core_axis_name="c", subcore_axis_name="s",
                                  num_cores=2)

    @functools.partial(
        pl.kernel,
        out_type=jax.ShapeDtypeStruct((_B, _NX * _NY), jnp.float32),
        mesh=mesh,
        compiler_params=pltpu.CompilerParams(needs_layout_passes=False),
        scratch_types=[
            pltpu.VMEM((_NSPOT, _NVAL), jnp.float32),
            pltpu.VMEM((_NSPOT, _NVAL), jnp.int32),
            pltpu.VMEM((_NX * _NY + 16,), jnp.float32),
            pltpu.SemaphoreType.DMA,
            pltpu.SemaphoreType.DMA,
        ],
    )
    def k(vals_hbm, idx_hbm, out_hbm, vals_v, idx_v, img_v, sem_v, sem_i):
        b = lax.axis_index("s") * 2 + lax.axis_index("c")
        cp_v = pltpu.async_copy(vals_hbm.at[pl.ds(b * _NSPOT, _NSPOT)],
                                vals_v, sem_v)
        cp_i = pltpu.async_copy(idx_hbm.at[pl.ds(b * _NSPOT, _NSPOT)],
                                idx_v, sem_i)

        @pl.loop(0, _NX * _NY + 16, step=16)
        def _zero(i):
            img_v[pl.ds(i, 16)] = jnp.zeros((16,), jnp.float32)

        cp_v.wait()
        cp_i.wait()

        @plsc.parallel_loop(0, _NSPOT, unroll=4)
        def _spot(s):
            for c in range(_NVAL // 16):
                ix = idx_v[s, pl.ds(c * 16, 16)]
                v = vals_v[s, pl.ds(c * 16, 16)]
                plsc.addupdate_scatter(img_v, [ix], v)

        pltpu.sync_copy(img_v.at[pl.ds(0, _NX * _NY)], out_hbm.at[b])

    return k(vals, idx)


def kernel(input, eps, W_mu1, b_mu1, W_mu2, b_mu2, W_mu3, b_mu3,
           W_lv1, b_lv1, W_lv2, b_lv2, W_lv3, b_lv3):
    x = input[:, 0]
    conv = pl.pallas_call(
        _blur_body,
        grid=(_B // _BIMG,),
        in_specs=[pl.BlockSpec((_BIMG, 256, 256), lambda b: (b, 0, 0)),
                  pl.BlockSpec((5, 260, 256), lambda b: (0, 0, 0))],
        out_specs=[pl.BlockSpec((_BIMG, 256, 256), lambda b: (b, 0, 0)),
                   pl.BlockSpec((_BIMG, _FC), lambda b: (b, 0))],
        out_shape=[jax.ShapeDtypeStruct((_B, 256, 256), jnp.float32),
                   jax.ShapeDtypeStruct((_B, _FC), jnp.float32)],
    )(x, jnp.asarray(_KC_BF))
    conv, act = conv

    fblk = _H1 // _NBLK
    mu, logvar, vals, idxs = pl.pallas_call(
        _mm_body,
        grid=(_NBLK,),
        in_specs=[
            pl.BlockSpec((_B, _FC), lambda i: (0, 0)),
            pl.BlockSpec((fblk, _FC), lambda i: (i, 0)),
            pl.BlockSpec((fblk, _FC), lambda i: (i, 0)),
            pl.BlockSpec((1, _H1), lambda i: (0, 0)),
            pl.BlockSpec((1, _H1), lambda i: (0, 0)),
            pl.BlockSpec((_H2, _H1), lambda i: (0, 0)),
            pl.BlockSpec((1, _H2), lambda i: (0, 0)),
            pl.BlockSpec((_LAT, _H2), lambda i: (0, 0)),
            pl.BlockSpec((1, _LAT), lambda i: (0, 0)),
            pl.BlockSpec((_H2, _H1), lambda i: (0, 0)),
            pl.BlockSpec((1, _H2), lambda i: (0, 0)),
            pl.BlockSpec((_LAT, _H2), lambda i: (0, 0)),
            pl.BlockSpec((1, _LAT), lambda i: (0, 0)),
            pl.BlockSpec((_B, _LAT), lambda i: (0, 0)),
        ],
        out_specs=[
            pl.BlockSpec((_B, _LAT), lambda i: (0, 0)),
            pl.BlockSpec((_B, _LAT), lambda i: (0, 0)),
            pl.BlockSpec((_B * _NSPOT, _NVAL), lambda i: (0, 0)),
            pl.BlockSpec((_B * _NSPOT, _NVAL), lambda i: (0, 0)),
        ],
        out_shape=[jax.ShapeDtypeStruct((_B, _LAT), jnp.float32),
                   jax.ShapeDtypeStruct((_B, _LAT), jnp.float32),
                   jax.ShapeDtypeStruct((_B * _NSPOT, _NVAL), jnp.float32),
                   jax.ShapeDtypeStruct((_B * _NSPOT, _NVAL), jnp.int32)],
        scratch_shapes=[
            pltpu.VMEM((_NBLK, _B, 2 * _H1 // _NBLK), jnp.float32),
        ],
    )(act, W_mu1, W_lv1,
      b_mu1.reshape(1, _H1), b_lv1.reshape(1, _H1),
      W_mu2, b_mu2.reshape(1, _H2), W_mu3, b_mu3.reshape(1, _LAT),
      W_lv2, b_lv2.reshape(1, _H2), W_lv3, b_lv3.reshape(1, _LAT), eps)

    img = _scatter_images(vals, idxs)
    return (img.reshape(_B, 1, _NX, _NY), conv.reshape(_B, 1, 256, 256),
            mu, logvar)
